# Initial kernel scaffold; baseline (speedup 1.0000x reference)
import jax
import jax.numpy as jnp
from jax.experimental import pallas as pl


def kernel(item_embeddings, rel1_idx, rel1_val, rel2_idx, rel2_val):
    raise NotImplementedError("write your pallas kernel here")


# speedup vs baseline: 3.9550x; 3.9550x over previous
import functools

import jax
import jax.numpy as jnp
from jax import lax
from jax.experimental import pallas as pl
from jax.experimental.pallas import tpu as pltpu
from jax.experimental.pallas import tpu_sc as plsc

N = 10000
D = 128
E = 160000
NR = 2
NS = 16
EPT = E // NS
K = 80
NG = K // 16
NCH = EPT // K
NRC = N // K
LANES = 16

_DNUMS = lax.GatherDimensionNumbers(
    offset_dims=(), collapsed_slice_dims=(0,), start_index_map=(0,))


def _splat(v16, e):
    idx = jnp.full((LANES, 1), e, jnp.int32)
    return lax.gather(v16, idx, _DNUMS, (1,),
                      mode=lax.GatherScatterMode.PROMISE_IN_BOUNDS)


def _sc_body(rows_hbm, cols_hbm, vals_hbm, emb_hbm,
             acc_out, deg_out0, deg_out1,
             row_buf, col_buf, val_buf, rows_v, acc, deg_acc):
    c = lax.axis_index("c")
    s = lax.axis_index("s")

    zeros16 = jnp.zeros((LANES,), jnp.float32)

    def zero_rows(i):
        for d in range(D // LANES):
            rows_v[i, pl.ds(d * LANES, LANES)] = zeros16
    pl.loop(0, K)(zero_rows)

    def zero_val(i):
        val_buf[pl.ds(i * LANES, LANES)] = zeros16
    pl.loop(0, NG)(zero_val)

    def acc_chunks(body):
        def wrapped(jj):
            j = s + jj * NS
            @pl.when(j < NRC)
            def _():
                body(j)
        pl.loop(0, (NRC + NS - 1) // NS)(wrapped)

    def zero_acc(j):
        pltpu.sync_copy(rows_v, acc.at[pl.ds(j * K, K)])
        pltpu.sync_copy(val_buf, deg_acc.at[pl.ds(j * K, K)])
    acc_chunks(zero_acc)

    plsc.subcore_barrier()

    edge_base = c * E + s * EPT

    def chunk(ch):
        base = edge_base + ch * K
        pltpu.sync_copy(rows_hbm.at[pl.ds(base, K)], row_buf)
        pltpu.sync_copy(cols_hbm.at[pl.ds(base, K)], col_buf)
        pltpu.sync_copy(vals_hbm.at[pl.ds(base, K)], val_buf)
        pltpu.sync_copy(emb_hbm.at[col_buf], rows_v)

        def scale(g):
            v16 = val_buf[pl.ds(g * LANES, LANES)]
            for e in range(LANES):
                vsplat = _splat(v16, e)
                r = g * LANES + e
                for d in range(D // LANES):
                    sl = pl.ds(d * LANES, LANES)
                    rows_v[r, sl] = rows_v[r, sl] * vsplat
        pl.loop(0, NG)(scale)

        pltpu.sync_copy(rows_v, acc.at[row_buf], add=True)
        pltpu.sync_copy(val_buf, deg_acc.at[row_buf], add=True)
    pl.loop(0, NCH)(chunk)

    plsc.subcore_barrier()

    def drain(j):
        pltpu.sync_copy(acc.at[pl.ds(j * K, K)], rows_v)
        pltpu.sync_copy(rows_v, acc_out.at[c, pl.ds(j * K, K)])
        pltpu.sync_copy(deg_acc.at[pl.ds(j * K, K)], val_buf)

        @pl.when(c == 0)
        def _():
            pltpu.sync_copy(val_buf, deg_out0.at[pl.ds(j * K, K)])

        @pl.when(c == 1)
        def _():
            pltpu.sync_copy(val_buf, deg_out1.at[pl.ds(j * K, K)])
    acc_chunks(drain)


_sc_agg = functools.partial(
    pl.kernel,
    out_type=[
        jax.ShapeDtypeStruct((NR, N, D), jnp.float32),
        jax.ShapeDtypeStruct((N,), jnp.float32),
        jax.ShapeDtypeStruct((N,), jnp.float32),
    ],
    mesh=plsc.VectorSubcoreMesh(core_axis_name="c", subcore_axis_name="s"),
    scratch_types=[
        pltpu.VMEM((K,), jnp.int32),
        pltpu.VMEM((K,), jnp.int32),
        pltpu.VMEM((K,), jnp.float32),
        pltpu.VMEM((K, D), jnp.float32),
        pltpu.VMEM_SHARED((N, D), jnp.float32),
        pltpu.VMEM_SHARED((N,), jnp.float32),
    ],
)(_sc_body)


def _combine_body(emb_ref, acc_ref, deg0_ref, deg1_ref, out_ref):
    d0 = jnp.maximum(deg0_ref[...], 1.0)
    d1 = jnp.maximum(deg1_ref[...], 1.0)
    p0 = acc_ref[0] / d0[:, None]
    p1 = acc_ref[1] / d1[:, None]
    out_ref[...] = emb_ref[...] + 0.5 * (p0 + p1)


def kernel(item_embeddings, rel1_idx, rel1_val, rel2_idx, rel2_val):
    rows = jnp.concatenate([rel1_idx[0], rel2_idx[0]])
    cols = jnp.concatenate([rel1_idx[1], rel2_idx[1]])
    vals = jnp.concatenate([rel1_val, rel2_val])

    acc, deg0, deg1 = _sc_agg(rows, cols, vals, item_embeddings)

    out = pl.pallas_call(
        _combine_body,
        out_shape=jax.ShapeDtypeStruct((N, D), jnp.float32),
    )(item_embeddings, acc, deg0, deg1)
    return out


# speedup vs baseline: 9.1715x; 2.3190x over previous
import functools

import jax
import jax.numpy as jnp
from jax import lax
from jax.experimental import pallas as pl
from jax.experimental.pallas import tpu as pltpu
from jax.experimental.pallas import tpu_sc as plsc

N = 10000
D = 128
E = 160000
NR = 2
NS = 16
EPT = E // NS
K = 80
NG = K // 16
NCH = EPT // K
NRC = N // K
LANES = 16

_DNUMS = lax.GatherDimensionNumbers(
    offset_dims=(), collapsed_slice_dims=(0,), start_index_map=(0,))


def _splat(v16, e):
    idx = jnp.full((LANES, 1), e, jnp.int32)
    return lax.gather(v16, idx, _DNUMS, (1,),
                      mode=lax.GatherScatterMode.PROMISE_IN_BOUNDS)


NIB = 4
NRB = 2


def _sc_body(rows_hbm, cols_hbm, vals_hbm, emb_hbm,
             acc_out, deg_out0, deg_out1,
             row_bufs, col_bufs, val_bufs, rows_vs, acc, deg_acc,
             sem_idx, sem_g, sem_sr, sem_sd):
    c = lax.axis_index("c")
    s = lax.axis_index("s")

    zeros16 = jnp.zeros((LANES,), jnp.float32)
    edge_base = c * E + s * EPT

    def issue_idx(j, t):
        base = edge_base + j * K
        pltpu.async_copy(rows_hbm.at[pl.ds(base, K)], row_bufs[t], sem_idx[t])
        pltpu.async_copy(cols_hbm.at[pl.ds(base, K)], col_bufs[t], sem_idx[t])
        pltpu.async_copy(vals_hbm.at[pl.ds(base, K)], val_bufs[t], sem_idx[t])

    def wait_idx(t):
        z = pl.ds(0, K)
        pltpu.make_async_copy(rows_hbm.at[z], row_bufs[t], sem_idx[t]).wait()
        pltpu.make_async_copy(cols_hbm.at[z], col_bufs[t], sem_idx[t]).wait()
        pltpu.make_async_copy(vals_hbm.at[z], val_bufs[t], sem_idx[t]).wait()

    def issue_gather(t, b):
        pltpu.async_copy(emb_hbm.at[col_bufs[t]], rows_vs[b], sem_g[b])

    def wait_gather(t, b):
        pltpu.make_async_copy(emb_hbm.at[col_bufs[t]], rows_vs[b],
                              sem_g[b]).wait()

    def issue_scatter(t, b):
        pltpu.async_copy(rows_vs[b], acc.at[row_bufs[t]], sem_sr[b], add=True)
        pltpu.async_copy(val_bufs[t], deg_acc.at[row_bufs[t]], sem_sd[t],
                         add=True)

    def wait_scatter_rows(b):
        pltpu.make_async_copy(rows_vs[b], acc.at[row_bufs[0]],
                              sem_sr[b]).wait()

    def wait_scatter_deg(t):
        pltpu.make_async_copy(val_bufs[t], deg_acc.at[row_bufs[0]],
                              sem_sd[t]).wait()

    def zero_rows(i):
        for d in range(D // LANES):
            rows_vs[0][i, pl.ds(d * LANES, LANES)] = zeros16
    pl.loop(0, K)(zero_rows)

    def zero_val(i):
        val_bufs[0][pl.ds(i * LANES, LANES)] = zeros16
    pl.loop(0, NG)(zero_val)

    def acc_chunks(body):
        def wrapped(jj):
            j = s + jj * NS
            @pl.when(j < NRC)
            def _():
                body(j)
        pl.loop(0, (NRC + NS - 1) // NS)(wrapped)

    def zero_acc(j):
        pltpu.sync_copy(rows_vs[0], acc.at[pl.ds(j * K, K)])
        pltpu.sync_copy(val_bufs[0], deg_acc.at[pl.ds(j * K, K)])
    acc_chunks(zero_acc)

    plsc.subcore_barrier()

    def scale(t, b):
        def body(g):
            v16 = val_bufs[t][pl.ds(g * LANES, LANES)]
            for e in range(LANES):
                vsplat = _splat(v16, e)
                r = g * LANES + e
                for d in range(D // LANES):
                    sl = pl.ds(d * LANES, LANES)
                    rows_vs[b][r, sl] = rows_vs[b][r, sl] * vsplat
        pl.loop(0, NG)(body)

    issue_idx(0, 0)
    issue_idx(1, 1)
    wait_idx(0)
    issue_gather(0, 0)

    def pipe(it):
        for u in range(NIB):
            i = it * NIB + u
            b = u % NRB

            @pl.when(i < NCH)
            def _():
                wait_gather(u, b)

                @pl.when(i >= 1)
                def _():
                    wait_scatter_rows((u + 1) % NRB)

                @pl.when(i + 1 < NCH)
                def _():
                    wait_idx((u + 1) % NIB)
                    issue_gather((u + 1) % NIB, (u + 1) % NRB)

                @pl.when(i >= 2)
                def _():
                    wait_scatter_deg((u + 2) % NIB)

                @pl.when(i + 2 < NCH)
                def _():
                    issue_idx(i + 2, (u + 2) % NIB)

                scale(u, b)
                issue_scatter(u, b)
    pl.loop(0, (NCH + NIB - 1) // NIB)(pipe)

    wait_scatter_rows((NCH - 1) % NRB)
    wait_scatter_deg((NCH - 2) % NIB)
    wait_scatter_deg((NCH - 1) % NIB)

    plsc.subcore_barrier()

    def drain(j):
        pltpu.sync_copy(acc.at[pl.ds(j * K, K)], rows_vs[0])
        pltpu.sync_copy(rows_vs[0], acc_out.at[c, pl.ds(j * K, K)])
        pltpu.sync_copy(deg_acc.at[pl.ds(j * K, K)], val_bufs[0])

        @pl.when(c == 0)
        def _():
            pltpu.sync_copy(val_bufs[0], deg_out0.at[pl.ds(j * K, K)])

        @pl.when(c == 1)
        def _():
            pltpu.sync_copy(val_bufs[0], deg_out1.at[pl.ds(j * K, K)])
    acc_chunks(drain)


_sc_agg = functools.partial(
    pl.kernel,
    out_type=[
        jax.ShapeDtypeStruct((NR, N, D), jnp.float32),
        jax.ShapeDtypeStruct((N,), jnp.float32),
        jax.ShapeDtypeStruct((N,), jnp.float32),
    ],
    mesh=plsc.VectorSubcoreMesh(core_axis_name="c", subcore_axis_name="s"),
    scratch_types=[
        [pltpu.VMEM((K,), jnp.int32)] * NIB,
        [pltpu.VMEM((K,), jnp.int32)] * NIB,
        [pltpu.VMEM((K,), jnp.float32)] * NIB,
        [pltpu.VMEM((K, D), jnp.float32)] * NRB,
        pltpu.VMEM_SHARED((N, D), jnp.float32),
        pltpu.VMEM_SHARED((N,), jnp.float32),
        [pltpu.SemaphoreType.DMA] * NIB,
        [pltpu.SemaphoreType.DMA] * NRB,
        [pltpu.SemaphoreType.DMA] * NRB,
        [pltpu.SemaphoreType.DMA] * NIB,
    ],
)(_sc_body)


def _combine_body(emb_ref, acc_ref, deg0_ref, deg1_ref, out_ref):
    d0 = jnp.maximum(deg0_ref[...], 1.0)
    d1 = jnp.maximum(deg1_ref[...], 1.0)
    p0 = acc_ref[0] / d0[:, None]
    p1 = acc_ref[1] / d1[:, None]
    out_ref[...] = emb_ref[...] + 0.5 * (p0 + p1)


def kernel(item_embeddings, rel1_idx, rel1_val, rel2_idx, rel2_val):
    rows = jnp.concatenate([rel1_idx[0], rel2_idx[0]])
    cols = jnp.concatenate([rel1_idx[1], rel2_idx[1]])
    vals = jnp.concatenate([rel1_val, rel2_val])

    acc, deg0, deg1 = _sc_agg(rows, cols, vals, item_embeddings)

    out = pl.pallas_call(
        _combine_body,
        out_shape=jax.ShapeDtypeStruct((N, D), jnp.float32),
    )(item_embeddings, acc, deg0, deg1)
    return out


# speedup vs baseline: 9.5639x; 1.0428x over previous
import functools

import jax
import jax.numpy as jnp
from jax import lax
from jax.experimental import pallas as pl
from jax.experimental.pallas import tpu as pltpu
from jax.experimental.pallas import tpu_sc as plsc

N = 10000
D = 128
E = 160000
NR = 2
NS = 16
K = 128
NCH = 78
MAIN = NCH * K
CPR = E // K
KT = 16
KR = 80
NRC = N // KR
LANES = 16

NIB = 4
NRB = 2

_DNUMS = lax.GatherDimensionNumbers(
    offset_dims=(), collapsed_slice_dims=(0,), start_index_map=(0,))

def _splat(v16, e):
    idx = jnp.full((LANES, 1), e, jnp.int32)
    return lax.gather(v16, idx, _DNUMS, (1,),
                      mode=lax.GatherScatterMode.PROMISE_IN_BOUNDS)


def _sc_body(icat_hbm, vals_hbm, rows_hbm, cols_hbm, emb_hbm,
             acc_out, deg_out0, deg_out1,
             ibufs, val_bufs, grows_vs,
             trow, tcol, tval, acc, deg_acc,
             sem_idx, sem_g, sem_sr, sem_sd):
    c = lax.axis_index("c")
    s = lax.axis_index("s")

    zeros16 = jnp.zeros((LANES,), jnp.float32)
    edge_base = c * E + s * MAIN
    chunk_base = c * CPR + s * NCH

    def issue_idx(j, t):
        pltpu.async_copy(icat_hbm.at[chunk_base + j], ibufs[t], sem_idx[t])
        pltpu.async_copy(vals_hbm.at[pl.ds(edge_base + j * K, K)],
                         val_bufs[t], sem_idx[t])

    def wait_idx(t):
        pltpu.make_async_copy(icat_hbm.at[0], ibufs[t], sem_idx[t]).wait()
        pltpu.make_async_copy(vals_hbm.at[pl.ds(0, K)], val_bufs[t],
                              sem_idx[t]).wait()

    def issue_gather(t, b):
        pltpu.async_copy(emb_hbm.at[ibufs[t].at[1]], grows_vs[b], sem_g[b])

    def wait_gather(b):
        pltpu.make_async_copy(emb_hbm.at[ibufs[0].at[1]], grows_vs[b],
                              sem_g[b]).wait()

    def issue_scatter(t, b):
        pltpu.async_copy(grows_vs[b], acc.at[ibufs[t].at[0]], sem_sr[b],
                         add=True)
        pltpu.async_copy(val_bufs[t], deg_acc.at[ibufs[t].at[0]], sem_sd[t],
                         add=True)

    def wait_scatter_rows(b):
        pltpu.make_async_copy(grows_vs[b], acc.at[ibufs[0].at[0]],
                              sem_sr[b]).wait()

    def wait_scatter_deg(t):
        pltpu.make_async_copy(val_bufs[t], deg_acc.at[ibufs[0].at[0]],
                              sem_sd[t]).wait()

    def zero_rows(i):
        for d in range(D // LANES):
            grows_vs[0][i, pl.ds(d * LANES, LANES)] = zeros16
    pl.loop(0, KR)(zero_rows)

    def zero_val(i):
        val_bufs[0][pl.ds(i * LANES, LANES)] = zeros16
    pl.loop(0, KR // LANES)(zero_val)

    def acc_chunks(body):
        def wrapped(jj):
            j = s + jj * NS
            @pl.when(j < NRC)
            def _():
                body(j)
        pl.loop(0, (NRC + NS - 1) // NS)(wrapped)

    zrows = grows_vs[0].at[pl.ds(0, KR)]
    zvals = val_bufs[0].at[pl.ds(0, KR)]

    def zero_acc(j):
        pltpu.sync_copy(zrows, acc.at[pl.ds(j * KR, KR)])
        pltpu.sync_copy(zvals, deg_acc.at[pl.ds(j * KR, KR)])
    acc_chunks(zero_acc)

    plsc.subcore_barrier()

    def expand_scale(e, b, vsplat, rows_buf, out_buf):
        for d in range(D // LANES):
            sl = pl.ds(d * LANES, LANES)
            out_buf[e, sl] = rows_buf[e, sl] * vsplat

    def scale(t, b):
        def body(g):
            v16 = val_bufs[t][pl.ds(g * LANES, LANES)]
            for e in range(LANES):
                vsplat = _splat(v16, e)
                expand_scale(g * LANES + e, b, vsplat,
                             grows_vs[b], grows_vs[b])
        pl.loop(0, K // LANES)(body)

    issue_idx(0, 0)
    issue_idx(1, 1)
    wait_idx(0)
    issue_gather(0, 0)

    def pipe(it):
        for u in range(NIB):
            i = it * NIB + u
            b = u % NRB

            @pl.when(i < NCH)
            def _():
                wait_gather(b)

                @pl.when(i >= 1)
                def _():
                    wait_scatter_rows((u + 1) % NRB)

                @pl.when(i + 1 < NCH)
                def _():
                    wait_idx((u + 1) % NIB)
                    issue_gather((u + 1) % NIB, (u + 1) % NRB)

                @pl.when(i >= 2)
                def _():
                    wait_scatter_deg((u + 2) % NIB)

                @pl.when(i + 2 < NCH)
                def _():
                    issue_idx(i + 2, (u + 2) % NIB)

                scale(u, b)
                issue_scatter(u, b)
    pl.loop(0, (NCH + NIB - 1) // NIB)(pipe)

    wait_scatter_rows((NCH - 1) % NRB)
    wait_scatter_deg((NCH - 2) % NIB)
    wait_scatter_deg((NCH - 1) % NIB)

    tbase = c * E + NS * MAIN + s * KT
    zt = pl.ds(0, KT)
    pltpu.async_copy(rows_hbm.at[pl.ds(tbase, KT)], trow, sem_idx[0])
    pltpu.async_copy(cols_hbm.at[pl.ds(tbase, KT)], tcol, sem_idx[0])
    pltpu.async_copy(vals_hbm.at[pl.ds(tbase, KT)], tval, sem_idx[0])
    pltpu.make_async_copy(rows_hbm.at[zt], trow, sem_idx[0]).wait()
    pltpu.make_async_copy(cols_hbm.at[zt], tcol, sem_idx[0]).wait()
    pltpu.make_async_copy(vals_hbm.at[zt], tval, sem_idx[0]).wait()
    tgrows = grows_vs[0].at[pl.ds(0, KT)]
    pltpu.async_copy(emb_hbm.at[tcol], tgrows, sem_g[0])
    pltpu.make_async_copy(emb_hbm.at[tcol], tgrows, sem_g[0]).wait()
    v16 = tval[pl.ds(0, LANES)]
    for e in range(LANES):
        vsplat = _splat(v16, e)
        expand_scale(e, 0, vsplat, grows_vs[0], grows_vs[0])
    tscaled = grows_vs[0].at[pl.ds(0, KT)]
    pltpu.async_copy(tscaled, acc.at[trow], sem_sr[0], add=True)
    pltpu.async_copy(tval, deg_acc.at[trow], sem_sd[0], add=True)
    pltpu.make_async_copy(tscaled, acc.at[trow], sem_sr[0]).wait()
    pltpu.make_async_copy(tval, deg_acc.at[trow], sem_sd[0]).wait()

    plsc.subcore_barrier()

    def drain(j):
        pltpu.sync_copy(acc.at[pl.ds(j * KR, KR)], zrows)
        pltpu.sync_copy(zrows, acc_out.at[c, pl.ds(j * KR, KR)])
        pltpu.sync_copy(deg_acc.at[pl.ds(j * KR, KR)], zvals)

        @pl.when(c == 0)
        def _():
            pltpu.sync_copy(zvals, deg_out0.at[pl.ds(j * KR, KR)])

        @pl.when(c == 1)
        def _():
            pltpu.sync_copy(zvals, deg_out1.at[pl.ds(j * KR, KR)])
    acc_chunks(drain)


_sc_agg = functools.partial(
    pl.kernel,
    out_type=[
        jax.ShapeDtypeStruct((NR, N, D), jnp.float32),
        jax.ShapeDtypeStruct((N,), jnp.float32),
        jax.ShapeDtypeStruct((N,), jnp.float32),
    ],
    mesh=plsc.VectorSubcoreMesh(core_axis_name="c", subcore_axis_name="s"),
    scratch_types=[
        [pltpu.VMEM((2, K), jnp.int32)] * NIB,
        [pltpu.VMEM((K,), jnp.float32)] * NIB,
        [pltpu.VMEM((K, D), jnp.float32)] * NRB,
        pltpu.VMEM((KT,), jnp.int32),
        pltpu.VMEM((KT,), jnp.int32),
        pltpu.VMEM((KT,), jnp.float32),
        pltpu.VMEM_SHARED((N, D), jnp.float32),
        pltpu.VMEM_SHARED((N,), jnp.float32),
        [pltpu.SemaphoreType.DMA] * NIB,
        [pltpu.SemaphoreType.DMA] * NRB,
        [pltpu.SemaphoreType.DMA] * NRB,
        [pltpu.SemaphoreType.DMA] * NIB,
    ],
)(_sc_body)


def _combine_body(emb_ref, acc_ref, deg0_ref, deg1_ref, out_ref):
    d0 = jnp.maximum(deg0_ref[...], 1.0)
    d1 = jnp.maximum(deg1_ref[...], 1.0)
    p0 = acc_ref[0] / d0[:, None]
    p1 = acc_ref[1] / d1[:, None]
    out_ref[...] = emb_ref[...] + 0.5 * (p0 + p1)


def kernel(item_embeddings, rel1_idx, rel1_val, rel2_idx, rel2_val):
    rows = jnp.concatenate([rel1_idx[0], rel2_idx[0]])
    cols = jnp.concatenate([rel1_idx[1], rel2_idx[1]])
    vals = jnp.concatenate([rel1_val, rel2_val])
    icat = jnp.stack([rows.reshape(-1, K), cols.reshape(-1, K)], axis=1)

    acc, deg0, deg1 = _sc_agg(icat, vals, rows, cols, item_embeddings)

    out = pl.pallas_call(
        _combine_body,
        out_shape=jax.ShapeDtypeStruct((N, D), jnp.float32),
    )(item_embeddings, acc, deg0, deg1)
    return out


# speedup vs baseline: 10.4138x; 1.0889x over previous
import functools

import jax
import jax.numpy as jnp
from jax import lax
from jax.experimental import pallas as pl
from jax.experimental.pallas import tpu as pltpu
from jax.experimental.pallas import tpu_sc as plsc

N = 10000
D = 128
E = 160000
NR = 2
NS = 16
K = 128
NCH = 78
MAIN = NCH * K
KT = 16
KR = 80
NRC = N // KR
LANES = 16

NIB = 4
NRB = 2

_DNUMS = lax.GatherDimensionNumbers(
    offset_dims=(), collapsed_slice_dims=(0,), start_index_map=(0,))


def _splat(v16, e):
    idx = jnp.full((LANES, 1), e, jnp.int32)
    return lax.gather(v16, idx, _DNUMS, (1,),
                      mode=lax.GatherScatterMode.PROMISE_IN_BOUNDS)


def _sc_body(rows_hbm, cols_hbm, vals_hbm, emb_hbm,
             acc_out, deg_out0, deg_out1,
             row_bufs, col_bufs, val_bufs, rows_vs,
             trow, tcol, tval, acc, deg_acc,
             sem_idx, sem_g, sem_sr, sem_sd):
    c = lax.axis_index("c")
    s = lax.axis_index("s")

    zeros16 = jnp.zeros((LANES,), jnp.float32)
    edge_base = c * E + s * MAIN

    def issue_idx(j, t):
        base = edge_base + j * K
        pltpu.async_copy(rows_hbm.at[pl.ds(base, K)], row_bufs[t], sem_idx[t])
        pltpu.async_copy(cols_hbm.at[pl.ds(base, K)], col_bufs[t], sem_idx[t])
        pltpu.async_copy(vals_hbm.at[pl.ds(base, K)], val_bufs[t], sem_idx[t])

    def wait_idx(t):
        z = pl.ds(0, K)
        pltpu.make_async_copy(rows_hbm.at[z], row_bufs[t], sem_idx[t]).wait()
        pltpu.make_async_copy(cols_hbm.at[z], col_bufs[t], sem_idx[t]).wait()
        pltpu.make_async_copy(vals_hbm.at[z], val_bufs[t], sem_idx[t]).wait()

    def issue_gather(t, b):
        pltpu.async_copy(emb_hbm.at[col_bufs[t]], rows_vs[b], sem_g[b])

    def wait_gather(b):
        pltpu.make_async_copy(emb_hbm.at[col_bufs[0]], rows_vs[b],
                              sem_g[b]).wait()

    def issue_scatter(t, b):
        pltpu.async_copy(rows_vs[b], acc.at[row_bufs[t]], sem_sr[b], add=True)
        pltpu.async_copy(val_bufs[t], deg_acc.at[row_bufs[t]], sem_sd[t],
                         add=True)

    def wait_scatter_rows(b):
        pltpu.make_async_copy(rows_vs[b], acc.at[row_bufs[0]],
                              sem_sr[b]).wait()

    def wait_scatter_deg(t):
        pltpu.make_async_copy(val_bufs[t], deg_acc.at[row_bufs[0]],
                              sem_sd[t]).wait()

    def zero_rows(i):
        for d in range(D // LANES):
            rows_vs[0][i, pl.ds(d * LANES, LANES)] = zeros16
    pl.loop(0, KR)(zero_rows)

    def zero_val(i):
        val_bufs[0][pl.ds(i * LANES, LANES)] = zeros16
    pl.loop(0, KR // LANES)(zero_val)

    zrows = rows_vs[0].at[pl.ds(0, KR)]
    zvals = val_bufs[0].at[pl.ds(0, KR)]

    def rc_loop(body):
        def wrapped(jj):
            j = s + jj * NS
            @pl.when(j < NRC)
            def _():
                body(j)
        pl.loop(0, (NRC + NS - 1) // NS)(wrapped)

    def zero_acc(j):
        pltpu.async_copy(zrows, acc.at[pl.ds(j * KR, KR)], sem_g[0])
        pltpu.async_copy(zvals, deg_acc.at[pl.ds(j * KR, KR)], sem_g[1])
    rc_loop(zero_acc)

    def zero_wait(j):
        pltpu.make_async_copy(zrows, acc.at[pl.ds(0, KR)], sem_g[0]).wait()
        pltpu.make_async_copy(zvals, deg_acc.at[pl.ds(0, KR)],
                              sem_g[1]).wait()
    rc_loop(zero_wait)

    plsc.subcore_barrier()

    def scale(t, b):
        def body(g):
            v16 = val_bufs[t][pl.ds(g * LANES, LANES)]
            for e in range(LANES):
                vsplat = _splat(v16, e)
                r = g * LANES + e
                for d in range(D // LANES):
                    sl = pl.ds(d * LANES, LANES)
                    rows_vs[b][r, sl] = rows_vs[b][r, sl] * vsplat
        pl.loop(0, K // LANES)(body)

    issue_idx(0, 0)
    issue_idx(1, 1)
    wait_idx(0)
    issue_gather(0, 0)

    def pipe(it):
        for u in range(NIB):
            i = it * NIB + u
            b = u % NRB

            @pl.when(i < NCH)
            def _():
                wait_gather(b)

                @pl.when(i >= 1)
                def _():
                    wait_scatter_rows((u + 1) % NRB)

                @pl.when(i + 1 < NCH)
                def _():
                    wait_idx((u + 1) % NIB)
                    issue_gather((u + 1) % NIB, (u + 1) % NRB)

                @pl.when(i >= 2)
                def _():
                    wait_scatter_deg((u + 2) % NIB)

                @pl.when(i + 2 < NCH)
                def _():
                    issue_idx(i + 2, (u + 2) % NIB)

                scale(u, b)
                issue_scatter(u, b)
    pl.loop(0, (NCH + NIB - 1) // NIB)(pipe)

    wait_scatter_rows((NCH - 1) % NRB)
    wait_scatter_deg((NCH - 2) % NIB)
    wait_scatter_deg((NCH - 1) % NIB)

    tbase = c * E + NS * MAIN + s * KT
    zt = pl.ds(0, KT)
    pltpu.async_copy(rows_hbm.at[pl.ds(tbase, KT)], trow, sem_idx[0])
    pltpu.async_copy(cols_hbm.at[pl.ds(tbase, KT)], tcol, sem_idx[0])
    pltpu.async_copy(vals_hbm.at[pl.ds(tbase, KT)], tval, sem_idx[0])
    pltpu.make_async_copy(rows_hbm.at[zt], trow, sem_idx[0]).wait()
    pltpu.make_async_copy(cols_hbm.at[zt], tcol, sem_idx[0]).wait()
    pltpu.make_async_copy(vals_hbm.at[zt], tval, sem_idx[0]).wait()
    tgrows = rows_vs[0].at[pl.ds(0, KT)]
    pltpu.async_copy(emb_hbm.at[tcol], tgrows, sem_g[0])
    pltpu.make_async_copy(emb_hbm.at[tcol], tgrows, sem_g[0]).wait()
    v16 = tval[pl.ds(0, LANES)]
    for e in range(LANES):
        vsplat = _splat(v16, e)
        for d in range(D // LANES):
            sl = pl.ds(d * LANES, LANES)
            rows_vs[0][e, sl] = rows_vs[0][e, sl] * vsplat
    pltpu.async_copy(tgrows, acc.at[trow], sem_sr[0], add=True)
    pltpu.async_copy(tval, deg_acc.at[trow], sem_sd[0], add=True)
    pltpu.make_async_copy(tgrows, acc.at[trow], sem_sr[0]).wait()
    pltpu.make_async_copy(tval, deg_acc.at[trow], sem_sd[0]).wait()

    plsc.subcore_barrier()

    def drain_in(j, b):
        pltpu.async_copy(acc.at[pl.ds(j * KR, KR)],
                         rows_vs[b].at[pl.ds(0, KR)], sem_g[b])

    def drain_in_wait(b):
        pltpu.make_async_copy(acc.at[pl.ds(0, KR)],
                              rows_vs[b].at[pl.ds(0, KR)], sem_g[b]).wait()

    def drain_out(j, b):
        pltpu.async_copy(rows_vs[b].at[pl.ds(0, KR)],
                         acc_out.at[c, pl.ds(j * KR, KR)], sem_sr[b])

    def drain_out_wait(b):
        pltpu.make_async_copy(rows_vs[b].at[pl.ds(0, KR)],
                              acc_out.at[c, pl.ds(0, KR)], sem_sr[b]).wait()

    NDC = (NRC + NS - 1) // NS

    def dj(k):
        return s + k * NS

    drain_in(dj(0), 0)

    def drain_step(kk):
        for h in range(2):
            k = kk * 2 + h
            b = h

            @pl.when(dj(k) < NRC)
            def _():
                drain_in_wait(b)

                @pl.when(dj(k + 1) < NRC)
                def _():
                    drain_in(dj(k + 1), 1 - b)

                @pl.when(k >= 2)
                def _():
                    drain_out_wait(b)

                drain_out(dj(k), b)
    pl.loop(0, NDC // 2)(drain_step)

    drain_out_wait(0)
    drain_out_wait(1)

    def drain_deg(j):
        pltpu.sync_copy(deg_acc.at[pl.ds(j * KR, KR)], zvals)

        @pl.when(c == 0)
        def _():
            pltpu.sync_copy(zvals, deg_out0.at[pl.ds(j * KR, KR)])

        @pl.when(c == 1)
        def _():
            pltpu.sync_copy(zvals, deg_out1.at[pl.ds(j * KR, KR)])
    rc_loop(drain_deg)


_sc_agg = functools.partial(
    pl.kernel,
    out_type=[
        jax.ShapeDtypeStruct((NR, N, D), jnp.float32),
        jax.ShapeDtypeStruct((N,), jnp.float32),
        jax.ShapeDtypeStruct((N,), jnp.float32),
    ],
    mesh=plsc.VectorSubcoreMesh(core_axis_name="c", subcore_axis_name="s"),
    scratch_types=[
        [pltpu.VMEM((K,), jnp.int32)] * NIB,
        [pltpu.VMEM((K,), jnp.int32)] * NIB,
        [pltpu.VMEM((K,), jnp.float32)] * NIB,
        [pltpu.VMEM((K, D), jnp.float32)] * NRB,
        pltpu.VMEM((KT,), jnp.int32),
        pltpu.VMEM((KT,), jnp.int32),
        pltpu.VMEM((KT,), jnp.float32),
        pltpu.VMEM_SHARED((N, D), jnp.float32),
        pltpu.VMEM_SHARED((N,), jnp.float32),
        [pltpu.SemaphoreType.DMA] * NIB,
        [pltpu.SemaphoreType.DMA] * NRB,
        [pltpu.SemaphoreType.DMA] * NRB,
        [pltpu.SemaphoreType.DMA] * NIB,
    ],
)(_sc_body)


def _combine_body(emb_ref, acc_ref, deg0_ref, deg1_ref, out_ref):
    d0 = jnp.maximum(deg0_ref[...], 1.0)
    d1 = jnp.maximum(deg1_ref[...], 1.0)
    p0 = acc_ref[0] / d0[:, None]
    p1 = acc_ref[1] / d1[:, None]
    out_ref[...] = emb_ref[...] + 0.5 * (p0 + p1)


def kernel(item_embeddings, rel1_idx, rel1_val, rel2_idx, rel2_val):
    rows = jnp.concatenate([rel1_idx[0], rel2_idx[0]])
    cols = jnp.concatenate([rel1_idx[1], rel2_idx[1]])
    vals = jnp.concatenate([rel1_val, rel2_val])

    acc, deg0, deg1 = _sc_agg(rows, cols, vals, item_embeddings)

    out = pl.pallas_call(
        _combine_body,
        out_shape=jax.ShapeDtypeStruct((N, D), jnp.float32),
    )(item_embeddings, acc, deg0, deg1)
    return out
